# baseline (speedup 1.0000x reference)
import functools

import jax
import jax.numpy as jnp
from jax import lax
from jax.experimental import pallas as pl
from jax.experimental.pallas import tpu as pltpu
from jax.experimental.pallas import tpu_sc as plsc

H = 8
NL, NR, EB, ERR, KN = 20000, 5000, 320000, 40000, 16
_INTERP = False
_DOT = jnp.dot


def _pad_up(n, m):
    return ((n + m - 1) // m) * m


def _gin_mlp(x_pad, agg_pad, w1p, b1, w2, b2):
    n, dinp = x_pad.shape
    dout = w2.shape[1]
    nt = n // 1000

    def body(x_ref, a_ref, w1_ref, b1_ref, w2_ref, b2_ref,
             y_ref, s_ref, q_ref):
        i = pl.program_id(0)
        hmid = jnp.maximum(_DOT(x_ref[...] + a_ref[...], w1_ref[...])
                           + b1_ref[...], 0.0)
        y = jnp.maximum(_DOT(hmid, w2_ref[...]) + b2_ref[...], 0.0)
        y_ref[...] = y

        @pl.when(i == 0)
        def _():
            s_ref[...] = jnp.zeros_like(s_ref)
            q_ref[...] = jnp.zeros_like(q_ref)

        s_ref[...] += jnp.sum(y, axis=0, keepdims=True)
        q_ref[...] += jnp.sum(y * y, axis=0, keepdims=True)

    full = lambda shape: pl.BlockSpec(shape, lambda i: (0, 0))
    return pl.pallas_call(
        body,
        grid=(nt,),
        in_specs=[pl.BlockSpec((1000, dinp), lambda i: (i, 0)),
                  pl.BlockSpec((1000, dinp), lambda i: (i, 0)),
                  full(w1p.shape), full(b1.shape),
                  full(w2.shape), full(b2.shape)],
        out_specs=[pl.BlockSpec((1000, dout), lambda i: (i, 0)),
                   full((1, dout)), full((1, dout))],
        out_shape=[jax.ShapeDtypeStruct((n, dout), jnp.float32),
                   jax.ShapeDtypeStruct((1, dout), jnp.float32),
                   jax.ShapeDtypeStruct((1, dout), jnp.float32)],
        interpret=_INTERP,
    )(x_pad, agg_pad, w1p, b1, w2, b2)


def _gat_feat(x_r, fc, al, ar, qp):
    n, din = x_r.shape
    dout = fc.shape[1] // H
    doutp = 4 * qp
    nt = n // 200

    def body(x_ref, fc_ref, al_ref, ar_ref,
             f0_ref, f1_ref, f2_ref, f3r_ref, el_ref, er_ref,
             ml_ref, mr_ref):
        i = pl.program_id(0)
        feat = _DOT(x_ref[...], fc_ref[...])
        f3 = feat.reshape(200, H, dout)
        el = jnp.sum(f3 * al_ref[...], axis=-1)
        er = jnp.sum(f3 * ar_ref[...], axis=-1)
        el_ref[...] = el
        er_ref[...] = er

        @pl.when(i == 0)
        def _():
            ml_ref[...] = jnp.full_like(ml_ref, -jnp.inf)
            mr_ref[...] = jnp.full_like(mr_ref, -jnp.inf)

        ml_ref[...] = jnp.maximum(ml_ref[...],
                                  jnp.max(el, axis=0, keepdims=True))
        mr_ref[...] = jnp.maximum(mr_ref[...],
                                  jnp.max(er, axis=0, keepdims=True))
        fp = jnp.concatenate(
            [f3, jnp.zeros((200, H, doutp - dout), jnp.float32)], axis=-1)
        for k, r in enumerate((f0_ref, f1_ref, f2_ref, f3r_ref)):
            r[...] = fp[:, :, k * qp:(k + 1) * qp]

    full = lambda shape: pl.BlockSpec(shape, lambda i: tuple(0 for _ in shape))
    qspec = pl.BlockSpec((200, H, qp), lambda i: (i, 0, 0))
    qshape = jax.ShapeDtypeStruct((n, H, qp), jnp.float32)
    return pl.pallas_call(
        body,
        grid=(nt,),
        in_specs=[pl.BlockSpec((200, din), lambda i: (i, 0)),
                  full(fc.shape), full((1, H, dout)), full((1, H, dout))],
        out_specs=[qspec, qspec, qspec, qspec,
                   pl.BlockSpec((200, H), lambda i: (i, 0)),
                   pl.BlockSpec((200, H), lambda i: (i, 0)),
                   full((1, H)), full((1, H))],
        out_shape=[qshape, qshape, qshape, qshape,
                   jax.ShapeDtypeStruct((n, H), jnp.float32),
                   jax.ShapeDtypeStruct((n, H), jnp.float32),
                   jax.ShapeDtypeStruct((1, H), jnp.float32),
                   jax.ShapeDtypeStruct((1, H), jnp.float32)],
        interpret=_INTERP,
    )(x_r, fc, al.reshape(1, H, dout), ar.reshape(1, H, dout))


def _lstm(neigh, wihp, whh, b):
    dinp = neigh.shape[1]
    din = whh.shape[0]
    rt = NR // 1000

    def body(nb_ref, wih_ref, whh_ref, b_ref, h_ref, h_scr, c_scr):
        t = pl.program_id(1)

        @pl.when(t == 0)
        def _():
            h_scr[...] = jnp.zeros_like(h_scr)
            c_scr[...] = jnp.zeros_like(c_scr)

        g = (_DOT(nb_ref[...], wih_ref[...]) + _DOT(h_scr[...], whh_ref[...])
             + b_ref[...])
        ig = jax.nn.sigmoid(g[:, :din])
        fg = jax.nn.sigmoid(g[:, din:2 * din])
        gg = jnp.tanh(g[:, 2 * din:3 * din])
        og = jax.nn.sigmoid(g[:, 3 * din:])
        c = fg * c_scr[...] + ig * gg
        hn = og * jnp.tanh(c)
        h_scr[...] = hn
        c_scr[...] = c
        h_ref[...] = hn

    full = lambda shape: pl.BlockSpec(shape, lambda r, t: tuple(0 for _ in shape))
    return pl.pallas_call(
        body,
        grid=(rt, KN),
        in_specs=[pl.BlockSpec((1000, dinp), lambda r, t: (t * rt + r, 0)),
                  full(wihp.shape), full(whh.shape), full(b.shape)],
        out_specs=pl.BlockSpec((1000, din), lambda r, t: (r, 0)),
        out_shape=jax.ShapeDtypeStruct((NR, din), jnp.float32),
        scratch_shapes=[pltpu.VMEM((1000, din), jnp.float32),
                        pltpu.VMEM((1000, din), jnp.float32)],
        interpret=_INTERP,
    )(neigh, wihp, whh, b)


def _sage_combine(x_r, hn, ws, wn, bsn, rst, cb):
    n, din = x_r.shape
    dout = ws.shape[1]
    nt = n // 1000

    def body(x_ref, h_ref, ws_ref, wn_ref, b_ref, r_ref, cb_ref,
             y_ref, s_ref, q_ref):
        i = pl.program_id(0)
        sage = (_DOT(x_ref[...], ws_ref[...]) + _DOT(h_ref[...], wn_ref[...])
                + b_ref[...])
        y = jnp.maximum(jnp.maximum(sage, 0.0) + r_ref[...] + cb_ref[...],
                        0.0)
        y_ref[...] = y

        @pl.when(i == 0)
        def _():
            s_ref[...] = jnp.zeros_like(s_ref)
            q_ref[...] = jnp.zeros_like(q_ref)

        s_ref[...] += jnp.sum(y, axis=0, keepdims=True)
        q_ref[...] += jnp.sum(y * y, axis=0, keepdims=True)

    full = lambda shape: pl.BlockSpec(shape, lambda i: (0, 0))
    return pl.pallas_call(
        body,
        grid=(nt,),
        in_specs=[pl.BlockSpec((1000, din), lambda i: (i, 0)),
                  pl.BlockSpec((1000, din), lambda i: (i, 0)),
                  full(ws.shape), full(wn.shape), full(bsn.shape),
                  pl.BlockSpec((1000, dout), lambda i: (i, 0)),
                  full(cb.shape)],
        out_specs=[pl.BlockSpec((1000, dout), lambda i: (i, 0)),
                   full((1, dout)), full((1, dout))],
        out_shape=[jax.ShapeDtypeStruct((n, dout), jnp.float32),
                   jax.ShapeDtypeStruct((1, dout), jnp.float32),
                   jax.ShapeDtypeStruct((1, dout), jnp.float32)],
        interpret=_INTERP,
    )(x_r, hn, ws, wn, bsn, rst, cb)


def _normalize_pad(y, s, t, dpad):
    n, dout = y.shape
    nt = n // 1000

    def body(y_ref, s_ref, t_ref, o_ref):
        v = y_ref[...] * s_ref[...] + t_ref[...]
        if dpad > dout:
            v = jnp.concatenate(
                [v, jnp.zeros((v.shape[0], dpad - dout), jnp.float32)],
                axis=-1)
        o_ref[...] = v

    full = lambda shape: pl.BlockSpec(shape, lambda i: (0, 0))
    return pl.pallas_call(
        body,
        grid=(nt,),
        in_specs=[pl.BlockSpec((1000, dout), lambda i: (i, 0)),
                  full((1, dout)), full((1, dout))],
        out_specs=pl.BlockSpec((1000, dpad), lambda i: (i, 0)),
        out_shape=jax.ShapeDtypeStruct((n, dpad), jnp.float32),
        interpret=_INTERP,
    )(y, s, t)


def _colmax(y):
    n, d = y.shape
    nt = n // 1000

    def body(y_ref, m_ref):
        i = pl.program_id(0)

        @pl.when(i == 0)
        def _():
            m_ref[...] = jnp.full_like(m_ref, -jnp.inf)

        m_ref[...] = jnp.maximum(m_ref[...],
                                 jnp.max(y_ref[...], axis=0, keepdims=True))

    return pl.pallas_call(
        body,
        grid=(nt,),
        in_specs=[pl.BlockSpec((1000, d), lambda i: (i, 0))],
        out_specs=pl.BlockSpec((1, d), lambda i: (0, 0)),
        out_shape=jax.ShapeDtypeStruct((1, d), jnp.float32),
        interpret=_INTERP,
    )(y)


def _head(lf, rf, w1a, w1b, b1, w2, b2, w3, b3):
    def body(lf_ref, rf_ref, w1a_ref, w1b_ref, b1_ref, w2_ref, b2_ref,
             w3_ref, b3_ref, o_ref):
        h1 = jnp.maximum(_DOT(lf_ref[...], w1a_ref[...])
                         + _DOT(rf_ref[...], w1b_ref[...]) + b1_ref[...], 0.0)
        h2 = jnp.maximum(_DOT(h1, w2_ref[...]) + b2_ref[...], 0.0)
        o_ref[...] = _DOT(h2, w3_ref[...]) + b3_ref[...]

    return pl.pallas_call(
        body,
        out_shape=jax.ShapeDtypeStruct((1, 1), jnp.float32),
        interpret=_INTERP,
    )(lf, rf, w1a, w1b, b1, w2, b2, w3, b3)


_MESH = plsc.VectorSubcoreMesh(core_axis_name="c", subcore_axis_name="s")


def _sc_gather_rows(x_pad, idx3):
    nw, nblk, bw = idx3.shape
    dinp = x_pad.shape[1]

    @functools.partial(
        pl.kernel,
        out_type=jax.ShapeDtypeStruct((nw * nblk * bw, dinp), jnp.float32),
        mesh=_MESH,
        scratch_types=[pltpu.VMEM((nblk, bw), jnp.int32),
                       pltpu.VMEM((bw, dinp), jnp.float32),
                       pltpu.VMEM((bw, dinp), jnp.float32),
                       pltpu.SemaphoreType.DMA,
                       pltpu.SemaphoreType.DMA],
    )
    def k(x_hbm, idx_hbm, out_hbm, idxb, rb0, rb1, s0, s1):
        wid = lax.axis_index("s") * 2 + lax.axis_index("c")
        pltpu.sync_copy(idx_hbm.at[wid], idxb)
        base = wid * (nblk * bw)
        bufs, sems = (rb0, rb1), (s0, s1)
        cps = {0: pltpu.async_copy(x_hbm.at[idxb.at[0]], rb0, s0)}
        for j in range(nblk):
            p = j % 2
            if j + 1 < nblk:
                cps[(j + 1) % 2] = pltpu.async_copy(
                    x_hbm.at[idxb.at[j + 1]], bufs[(j + 1) % 2],
                    sems[(j + 1) % 2])
            cps[p].wait()
            pltpu.sync_copy(bufs[p], out_hbm.at[pl.ds(base + j * bw, bw)])

    return k(x_pad, idx3)


def _sc_segment_sum(x_flat, srcf3, dst3, n_grp):
    _, nsb, sbe = srcf3.shape
    nbl = sbe // 128
    nlp = NL + 16
    nrow = nlp // 16

    @functools.partial(
        pl.kernel,
        out_type=jax.ShapeDtypeStruct((nlp, n_grp, 64), jnp.float32),
        mesh=_MESH,
        scratch_types=[pltpu.VMEM((sbe,), jnp.int32),
                       pltpu.VMEM((nbl, 128), jnp.int32),
                       pltpu.VMEM((128,), jnp.int32),
                       pltpu.VMEM((128,), jnp.int32),
                       pltpu.VMEM((128, 64), jnp.float32),
                       pltpu.VMEM((128, 64), jnp.float32),
                       pltpu.VMEM_SHARED((nlp, 64), jnp.float32),
                       pltpu.SemaphoreType.DMA,
                       pltpu.SemaphoreType.DMA],
        compiler_params=pltpu.CompilerParams(use_tc_tiling_on_sc=False),
    )
    def k(x_hbm, src_hbm, dst_hbm, out_hbm,
          srcf, dstb, gx0, gx1, rb0, rb1, acc, s0, s1):
        cid = lax.axis_index("c")
        sid = lax.axis_index("s")
        zv = jnp.zeros((16,), jnp.float32)

        def zfill(i, _):
            for q in range(4):
                rb0[i, pl.ds(16 * q, 16)] = zv
            return 0

        lo = sid * nrow
        gxs, bufs, sems = (gx0, gx1), (rb0, rb1), (s0, s1)
        for gi in range(n_grp // 2):
            g = gi * 2 + cid
            lax.fori_loop(0, 128, zfill, 0)
            nfull = nrow // 128
            for j in range(nfull):
                pltpu.sync_copy(rb0, acc.at[pl.ds(lo + j * 128, 128)])
            rem = nrow - nfull * 128
            if rem:
                pltpu.sync_copy(rb0.at[pl.ds(0, rem)],
                                acc.at[pl.ds(lo + nfull * 128, rem)])
            plsc.subcore_barrier()

            def sbody(sb, _):
                pltpu.sync_copy(src_hbm.at[sid, sb], srcf)
                pltpu.sync_copy(dst_hbm.at[sid, sb], dstb)

                def fire(j, p):
                    def gfill(kk, _):
                        sv = srcf[pl.ds(j * 128 + kk * 16, 16)]
                        gxs[p][pl.ds(kk * 16, 16)] = sv * n_grp + g
                        return 0

                    lax.fori_loop(0, 8, gfill, 0)
                    return pltpu.async_copy(x_hbm.at[gxs[p]], bufs[p],
                                            sems[p])

                cps = {0: fire(0, 0)}
                for j in range(nbl):
                    p = j % 2
                    cps[p].wait()
                    if j + 1 < nbl:
                        cps[(j + 1) % 2] = fire(j + 1, (j + 1) % 2)
                    pltpu.sync_copy(bufs[p], acc.at[dstb.at[j]], add=True)
                return 0

            lax.fori_loop(0, nsb, sbody, 0)
            plsc.subcore_barrier()
            nfull = nrow // 128
            for j in range(nfull):
                pltpu.sync_copy(acc.at[pl.ds(lo + j * 128, 128)],
                                out_hbm.at[pl.ds(lo + j * 128, 128), g])
            rem = nrow - nfull * 128
            if rem:
                pltpu.sync_copy(acc.at[pl.ds(lo + nfull * 128, rem)],
                                out_hbm.at[pl.ds(lo + nfull * 128, rem), g])
            plsc.subcore_barrier()

    return k(x_flat, srcf3, dst3)


def _sc_gat_pass1(el2, er2, m16, srcp1, dstp1, nerrp):
    nt = nerrp // 16
    nv = nt // 16

    @functools.partial(
        pl.kernel,
        out_type=[jax.ShapeDtypeStruct((H, nerrp), jnp.float32),
                  jax.ShapeDtypeStruct((2, 384, 64), jnp.float32)],
        mesh=_MESH,
        scratch_types=[pltpu.VMEM((NR + 16, 4), jnp.float32),
                       pltpu.VMEM((NR + 16, 4), jnp.float32),
                       pltpu.VMEM((128,), jnp.float32),
                       pltpu.VMEM((nt,), jnp.int32),
                       pltpu.VMEM((nt,), jnp.int32),
                       pltpu.VMEM((4, nt), jnp.float32),
                       pltpu.VMEM((384, 64), jnp.float32),
                       pltpu.VMEM((3, 128), jnp.int32),
                       pltpu.VMEM_SHARED((384, 64), jnp.float32)],
        compiler_params=pltpu.CompilerParams(use_tc_tiling_on_sc=False, needs_layout_passes=False),
    )
    def k(el_hbm, er_hbm, m_hbm, src_hbm, dst_hbm, ex_hbm, den_hbm,
          elb, erb, mb, srcf, dstf, exb, den, ident, den_sh):
        cid = lax.axis_index("c")
        sid = lax.axis_index("s")
        pltpu.sync_copy(el_hbm.at[cid], elb.at[pl.ds(0, NR)])
        pltpu.sync_copy(er_hbm.at[cid], erb.at[pl.ds(0, NR)])
        pltpu.sync_copy(m_hbm, mb)
        pltpu.sync_copy(src_hbm.at[sid], srcf)
        pltpu.sync_copy(dst_hbm.at[sid], dstf)
        zv = jnp.zeros((16,), jnp.float32)
        iot = lax.iota(jnp.int32, 16)

        def dfill(i, _):
            for q in range(4):
                den[i, pl.ds(16 * q, 16)] = zv
            return 0

        lax.fori_loop(0, 384, dfill, 0)

        def ifill(i, _):
            ident[i >> 3, pl.ds((i & 7) * 16, 16)] = iot + i * 16
            return 0

        lax.fori_loop(0, 24, ifill, 0)
        pltpu.sync_copy(den.at[pl.ds(0, 24)], den_sh.at[pl.ds(sid * 24, 24)])
        plsc.subcore_barrier()

        def ebody(j, _):
            sv = srcf[pl.ds(16 * j, 16)]
            dv = dstf[pl.ds(16 * j, 16)]
            for hh in range(4):
                hv = jnp.full((16,), hh, jnp.int32)
                li = plsc.load_gather(elb, [sv, hv])
                ri = plsc.load_gather(erb, [dv, hv])
                e = li + ri
                e = jnp.where(e >= 0, e, 0.2 * e)
                mv = mb[pl.ds((4 * cid + hh) * 16, 16)]
                ex = jnp.exp(e - mv)
                exb[hh, pl.ds(16 * j, 16)] = ex
                flat = dv * 4 + hh
                plsc.addupdate_scatter(
                    den, [lax.shift_right_logical(flat, 6), flat & 63], ex)
            return 0

        lax.fori_loop(0, nv, ebody, 0)
        for j in range(3):
            pltpu.sync_copy(den.at[pl.ds(128 * j, 128)],
                            den_sh.at[ident.at[j]], add=True)
        plsc.subcore_barrier()
        for hh in range(4):
            pltpu.sync_copy(exb.at[hh],
                            ex_hbm.at[4 * cid + hh, pl.ds(sid * nt, nt)])
        pltpu.sync_copy(den_sh.at[pl.ds(sid * 24, 24)],
                        den_hbm.at[cid, pl.ds(sid * 24, 24)])

    return k(el2, er2, m16, srcp1, dstp1)


def _sc_gat_pass2(feat_flat, den_pad, ex, srcp1, dst2p, qp):
    nerrp = ex.shape[1]
    nt = nerrp // 16
    nbl = nt // 128
    fw = feat_flat.shape[1]
    hq16 = qp // 16

    @functools.partial(
        pl.kernel,
        out_type=jax.ShapeDtypeStruct((4, 5120, qp), jnp.float32),
        mesh=_MESH,
        scratch_types=[pltpu.VMEM((nt,), jnp.int32),
                       pltpu.VMEM((nt,), jnp.int32),
                       pltpu.VMEM((20, 128), jnp.int32),
                       pltpu.VMEM((1024,), jnp.float32),
                       pltpu.VMEM((128, 16), jnp.float32),
                       pltpu.VMEM((8, fw), jnp.float32),
                       pltpu.VMEM((8, fw), jnp.float32),
                       pltpu.VMEM((128, qp), jnp.float32),
                       pltpu.VMEM_SHARED((5120, qp), jnp.float32),
                       pltpu.SemaphoreType.DMA,
                       pltpu.SemaphoreType.DMA,
                       pltpu.SemaphoreType.DMA],
        compiler_params=pltpu.CompilerParams(use_tc_tiling_on_sc=False, needs_layout_passes=False),
    )
    def k(f_hbm, den_hbm, ex_hbm, src_hbm, dst2_hbm, out_hbm,
          srcf, srcg, dst2d, exb, denb, fb0, fb1, wbuf, rst_sh, sA, s0, s1):
        cid = lax.axis_index("c")
        sid = lax.axis_index("s")
        pltpu.sync_copy(src_hbm.at[sid], srcf)
        pltpu.sync_copy(dst2_hbm.at[sid], dst2d)
        iot = lax.iota(jnp.int32, 16)
        zv = jnp.zeros((16,), jnp.float32)
        fbs, sems = (fb0, fb1), (s0, s1)
        lo = sid * 320

        def alph(kk, _):
            idxv = iot + 16 * kk
            for h in range(H):
                hv = jnp.full((16,), h, jnp.int32)
                dg = plsc.load_gather(denb, [idxv, hv])
                a = exb[pl.ds(h * 128 + 16 * kk, 16)] / ((dg + 1e-9) * 8.0)
                exb[pl.ds(h * 128 + 16 * kk, 16)] = a
            return 0

        def zfill(i, _):
            iv = jnp.full((16,), i, jnp.int32)
            for q in range(hq16):
                plsc.store_scatter(wbuf, [iv, iot + 16 * q], zv)
            return 0

        for gi in range(2):
            qg = gi * 2 + cid

            def soff(j, _):
                srcg[pl.ds(16 * j, 16)] = srcf[pl.ds(16 * j, 16)] + NR * qg
                return 0

            lax.fori_loop(0, nt // 16, soff, 0)
            lax.fori_loop(0, 128, zfill, 0)
            pltpu.sync_copy(wbuf, rst_sh.at[pl.ds(lo, 128)])
            pltpu.sync_copy(wbuf, rst_sh.at[pl.ds(lo + 128, 128)])
            pltpu.sync_copy(wbuf.at[pl.ds(0, 64)],
                            rst_sh.at[pl.ds(lo + 256, 64)])
            plsc.subcore_barrier()

            def sub(jb, sq, p):
                for ee in range(8):
                    col = sq * 8 + ee
                    colv = jnp.full((16,), col, jnp.int32)
                    avs = [plsc.load_gather(
                        exb, [jnp.full((16,), h * 128 + col, jnp.int32)])
                        for h in range(H)]
                    eev = jnp.full((16,), ee, jnp.int32)
                    for q in range(hq16):
                        acc = avs[0] * plsc.load_gather(
                            fbs[p], [eev, iot + q * 16])
                        for h in range(1, H):
                            acc += avs[h] * plsc.load_gather(
                                fbs[p], [eev, iot + h * qp + q * 16])
                        plsc.store_scatter(wbuf, [colv, iot + q * 16], acc)

            def fire(jb, sq, p):
                sqc = jnp.minimum(sq, 15)
                return pltpu.async_copy(
                    f_hbm.at[srcg.at[pl.ds(jb * 128 + sqc * 8, 8)]],
                    fbs[p], sems[p])

            def drain(p):
                pltpu.make_async_copy(f_hbm.at[pl.ds(0, 8)], fbs[p],
                                      sems[p]).wait()

            def block(jb, _):
                pltpu.async_copy(den_hbm.at[dst2d.at[jb]], denb, sA).wait()
                for h in range(H):
                    pltpu.sync_copy(
                        ex_hbm.at[h, pl.ds(sid * nt + jb * 128, 128)],
                        exb.at[pl.ds(h * 128, 128)])
                lax.fori_loop(0, 8, alph, 0)
                fire(jb, 0, 0)

                def pair(qq, _):
                    fire(jb, 2 * qq + 1, 1)
                    drain(0)
                    sub(jb, 2 * qq, 0)
                    fire(jb, 2 * qq + 2, 0)
                    drain(1)
                    sub(jb, 2 * qq + 1, 1)
                    return 0

                lax.fori_loop(0, 8, pair, 0)
                drain(0)
                pltpu.sync_copy(wbuf, rst_sh.at[dst2d.at[jb]], add=True)
                return 0

            lax.fori_loop(0, nbl, block, 0)
            plsc.subcore_barrier()
            pltpu.sync_copy(rst_sh.at[pl.ds(lo, 128)],
                            out_hbm.at[qg, pl.ds(lo, 128)])
            pltpu.sync_copy(rst_sh.at[pl.ds(lo + 128, 128)],
                            out_hbm.at[qg, pl.ds(lo + 128, 128)])
            pltpu.sync_copy(rst_sh.at[pl.ds(lo + 256, 64)],
                            out_hbm.at[qg, pl.ds(lo + 256, 64)])
            plsc.subcore_barrier()

    return k(feat_flat, den_pad, ex, srcp1, dst2p)


def _sage_gather(x_pad, gidx):
    return x_pad[gidx]


def _gin_agg(x_pad, src, dst):
    return jax.ops.segment_sum(x_pad[src], dst, num_segments=NL)


def _gat_edges(el, er, mvec, feat_a, feat_b, src, dst, dout, halfp):
    half = dout // 2
    e = el[src] + er[dst]
    e = jnp.where(e >= 0, e, 0.2 * e)
    ex = jnp.exp(e - mvec[None, :])
    den = jax.ops.segment_sum(ex, dst, num_segments=NR)
    alpha = ex / (den[dst] + 1e-9) / H
    feat = jnp.concatenate([feat_a[:, :, :half], feat_b[:, :, :half]],
                           axis=-1)
    w = jnp.sum(feat[src] * alpha[:, :, None], axis=1)
    return jax.ops.segment_sum(w, dst, num_segments=NR)


def _finalize_bn(ssum, ssq, gamma, beta, n):
    mean = ssum / n
    var = ssq / n - mean * mean
    s = gamma.reshape(1, -1) / jnp.sqrt(var + 1e-5)
    t = beta.reshape(1, -1) - mean * s
    return s, t


def kernel(x_ligand, x_residue, bond_edge_index, rr_edge_index,
           lr_neighbors, params):
    src_b, dst_b = bond_edge_index[0], bond_edge_index[1]
    src_r, dst_r = rr_edge_index[0], rr_edge_index[1]
    lrn_t = jnp.swapaxes(lr_neighbors, 0, 1).reshape(-1)

    ebp = 16 * 10 * 2048
    srcb3 = jnp.pad(src_b, (0, ebp - EB)).reshape(16, 10, 2048)
    dstb3 = jnp.pad(dst_b, (0, ebp - EB),
                    constant_values=NL).reshape(16, 10, 16, 128)
    idx3 = jnp.pad(lrn_t, (0, 32 * 20 * 128 - KN * NR)).reshape(32, 20, 128)
    errp = 16 * 2560
    srcp1 = jnp.pad(src_r, (0, errp - ERR)).reshape(16, 2560)
    dstp1 = jnp.pad(dst_r, (0, errp - ERR), constant_values=NR).reshape(16, 2560)
    dst2p = dstp1.reshape(16, 20, 128)

    dims = [(108, 108), (108, 216), (216, 432)]
    dinp0 = _pad_up(108, 64)
    x_l = jnp.pad(x_ligand, ((0, 0), (0, dinp0 - 108)))
    x_r = x_residue

    for li, lp in enumerate(params["layers"]):
        din, dout = dims[li]
        dinp = _pad_up(din, 64)
        doutp = _pad_up(dout, 64)
        qp = doutp // 4

        n_grp = dinp // 64
        agg = _sc_segment_sum(x_l.reshape(NL * n_grp, 64), srcb3, dstb3,
                              n_grp)[:NL].reshape(NL, dinp)
        w1p = jnp.pad(lp["gin"]["l1"]["W"], ((0, dinp - din), (0, 0)))
        y_l, sl, ql = _gin_mlp(x_l, agg, w1p,
                               lp["gin"]["l1"]["b"].reshape(1, -1),
                               lp["gin"]["l2"]["W"],
                               lp["gin"]["l2"]["b"].reshape(1, -1))

        f0, f1, f2, f3s, el, er, ml, mr = _gat_feat(x_r, lp["gat"]["fc"],
                                                    lp["gat"]["attn_l"],
                                                    lp["gat"]["attn_r"], qp)
        msum = (ml + mr).reshape(-1)
        mvec = jnp.where(msum >= 0, msum, 0.2 * msum)
        m16 = jnp.repeat(mvec, 16)
        el2 = el.reshape(NR, 2, 4).transpose(1, 0, 2)
        er2 = er.reshape(NR, 2, 4).transpose(1, 0, 2)
        ex, den2 = _sc_gat_pass1(el2, er2, m16, srcp1, dstp1, errp)
        den = (den2.reshape(2, 24576)[:, :NR * 4].reshape(2, NR, 4)
               .transpose(1, 0, 2).reshape(NR, H))
        den_pad = jnp.pad(den, ((0, 5120 - NR), (0, 8)))
        featf = jnp.stack([f0, f1, f2, f3s]).reshape(4 * NR, H * qp)
        rst4 = _sc_gat_pass2(featf, den_pad, ex, srcp1, dst2p, qp)
        rst = jnp.concatenate([rst4[q][:NR] for q in range(4)],
                              axis=1)[:, :dout]

        neigh = _sc_gather_rows(x_l, idx3)
        wihp = jnp.pad(lp["sage"]["W_ih"], ((0, dinp - din), (0, 0)))
        bsum = (lp["sage"]["b_ih"] + lp["sage"]["b_hh"]).reshape(1, -1)
        hn = _lstm(neigh, wihp, lp["sage"]["W_hh"], bsum)

        cb = jnp.mean(lp["gat"]["bias"], axis=0).reshape(1, -1)
        bsn = (lp["sage"]["fc_self"]["b"]
               + lp["sage"]["fc_neigh"]["b"]).reshape(1, -1)
        y_r, sr, qr = _sage_combine(x_r, hn, lp["sage"]["fc_self"]["W"],
                                    lp["sage"]["fc_neigh"]["W"], bsn, rst, cb)

        s_l, t_l = _finalize_bn(sl, ql, lp["bn_l"]["gamma"],
                                lp["bn_l"]["beta"], NL)
        s_r, t_r = _finalize_bn(sr, qr, lp["bn_r"]["gamma"],
                                lp["bn_r"]["beta"], NR)
        if li < 2:
            dnext = _pad_up(dims[li + 1][0], 64)
            x_l = _normalize_pad(y_l, s_l, t_l, dnext)
            x_r = _normalize_pad(y_r, s_r, t_r, dout)
        else:
            lig_feat = _colmax(y_l) * s_l + t_l
            res_feat = _colmax(y_r) * s_r + t_r

    x3 = _head(lig_feat, res_feat,
               params["fc01"]["W"][:432], params["fc01"]["W"][432:],
               params["fc01"]["b"].reshape(1, -1),
               params["fc02"]["W"], params["fc02"]["b"].reshape(1, -1),
               params["fc03"]["W"], params["fc03"]["b"].reshape(1, -1))
    return (lig_feat, res_feat, x3)


# speedup vs baseline: 1.5019x; 1.5019x over previous
import functools

import jax
import jax.numpy as jnp
from jax import lax
from jax.experimental import pallas as pl
from jax.experimental.pallas import tpu as pltpu
from jax.experimental.pallas import tpu_sc as plsc

H = 8
NL, NR, EB, ERR, KN = 20000, 5000, 320000, 40000, 16
_INTERP = False
_DOT = jnp.dot


def _pad_up(n, m):
    return ((n + m - 1) // m) * m


def _gin_mlp(x_pad, agg_pad, w1p, b1, w2, b2):
    n, dinp = x_pad.shape
    dout = w2.shape[1]
    nt = n // 1000

    def body(x_ref, a_ref, w1_ref, b1_ref, w2_ref, b2_ref,
             y_ref, s_ref, q_ref):
        i = pl.program_id(0)
        hmid = jnp.maximum(_DOT(x_ref[...] + a_ref[...], w1_ref[...])
                           + b1_ref[...], 0.0)
        y = jnp.maximum(_DOT(hmid, w2_ref[...]) + b2_ref[...], 0.0)
        y_ref[...] = y

        @pl.when(i == 0)
        def _():
            s_ref[...] = jnp.zeros_like(s_ref)
            q_ref[...] = jnp.zeros_like(q_ref)

        s_ref[...] += jnp.sum(y, axis=0, keepdims=True)
        q_ref[...] += jnp.sum(y * y, axis=0, keepdims=True)

    full = lambda shape: pl.BlockSpec(shape, lambda i: (0, 0))
    return pl.pallas_call(
        body,
        grid=(nt,),
        in_specs=[pl.BlockSpec((1000, dinp), lambda i: (i, 0)),
                  pl.BlockSpec((1000, dinp), lambda i: (i, 0)),
                  full(w1p.shape), full(b1.shape),
                  full(w2.shape), full(b2.shape)],
        out_specs=[pl.BlockSpec((1000, dout), lambda i: (i, 0)),
                   full((1, dout)), full((1, dout))],
        out_shape=[jax.ShapeDtypeStruct((n, dout), jnp.float32),
                   jax.ShapeDtypeStruct((1, dout), jnp.float32),
                   jax.ShapeDtypeStruct((1, dout), jnp.float32)],
        interpret=_INTERP,
    )(x_pad, agg_pad, w1p, b1, w2, b2)


def _gat_feat(x_r, fc, al, ar, qp):
    n, din = x_r.shape
    dout = fc.shape[1] // H
    doutp = 4 * qp
    nt = n // 200

    def body(x_ref, fc_ref, al_ref, ar_ref,
             f0_ref, f1_ref, f2_ref, f3r_ref, el_ref, er_ref,
             ml_ref, mr_ref):
        i = pl.program_id(0)
        feat = _DOT(x_ref[...], fc_ref[...])
        f3 = feat.reshape(200, H, dout)
        el = jnp.sum(f3 * al_ref[...], axis=-1)
        er = jnp.sum(f3 * ar_ref[...], axis=-1)
        el_ref[...] = el
        er_ref[...] = er

        @pl.when(i == 0)
        def _():
            ml_ref[...] = jnp.full_like(ml_ref, -jnp.inf)
            mr_ref[...] = jnp.full_like(mr_ref, -jnp.inf)

        ml_ref[...] = jnp.maximum(ml_ref[...],
                                  jnp.max(el, axis=0, keepdims=True))
        mr_ref[...] = jnp.maximum(mr_ref[...],
                                  jnp.max(er, axis=0, keepdims=True))
        fp = jnp.concatenate(
            [f3, jnp.zeros((200, H, doutp - dout), jnp.float32)], axis=-1)
        for k, r in enumerate((f0_ref, f1_ref, f2_ref, f3r_ref)):
            r[...] = fp[:, :, k * qp:(k + 1) * qp]

    full = lambda shape: pl.BlockSpec(shape, lambda i: tuple(0 for _ in shape))
    qspec = pl.BlockSpec((200, H, qp), lambda i: (i, 0, 0))
    qshape = jax.ShapeDtypeStruct((n, H, qp), jnp.float32)
    return pl.pallas_call(
        body,
        grid=(nt,),
        in_specs=[pl.BlockSpec((200, din), lambda i: (i, 0)),
                  full(fc.shape), full((1, H, dout)), full((1, H, dout))],
        out_specs=[qspec, qspec, qspec, qspec,
                   pl.BlockSpec((200, H), lambda i: (i, 0)),
                   pl.BlockSpec((200, H), lambda i: (i, 0)),
                   full((1, H)), full((1, H))],
        out_shape=[qshape, qshape, qshape, qshape,
                   jax.ShapeDtypeStruct((n, H), jnp.float32),
                   jax.ShapeDtypeStruct((n, H), jnp.float32),
                   jax.ShapeDtypeStruct((1, H), jnp.float32),
                   jax.ShapeDtypeStruct((1, H), jnp.float32)],
        interpret=_INTERP,
    )(x_r, fc, al.reshape(1, H, dout), ar.reshape(1, H, dout))


def _lstm(neigh, wihp, whh, b):
    dinp = neigh.shape[1]
    din = whh.shape[0]
    rt = NR // 1000

    def body(nb_ref, wih_ref, whh_ref, b_ref, h_ref, h_scr, c_scr):
        t = pl.program_id(1)

        @pl.when(t == 0)
        def _():
            h_scr[...] = jnp.zeros_like(h_scr)
            c_scr[...] = jnp.zeros_like(c_scr)

        g = (_DOT(nb_ref[...], wih_ref[...]) + _DOT(h_scr[...], whh_ref[...])
             + b_ref[...])
        ig = jax.nn.sigmoid(g[:, :din])
        fg = jax.nn.sigmoid(g[:, din:2 * din])
        gg = jnp.tanh(g[:, 2 * din:3 * din])
        og = jax.nn.sigmoid(g[:, 3 * din:])
        c = fg * c_scr[...] + ig * gg
        hn = og * jnp.tanh(c)
        h_scr[...] = hn
        c_scr[...] = c
        h_ref[...] = hn

    full = lambda shape: pl.BlockSpec(shape, lambda r, t: tuple(0 for _ in shape))
    return pl.pallas_call(
        body,
        grid=(rt, KN),
        in_specs=[pl.BlockSpec((1000, dinp), lambda r, t: (t * rt + r, 0)),
                  full(wihp.shape), full(whh.shape), full(b.shape)],
        out_specs=pl.BlockSpec((1000, din), lambda r, t: (r, 0)),
        out_shape=jax.ShapeDtypeStruct((NR, din), jnp.float32),
        scratch_shapes=[pltpu.VMEM((1000, din), jnp.float32),
                        pltpu.VMEM((1000, din), jnp.float32)],
        interpret=_INTERP,
    )(neigh, wihp, whh, b)


def _sage_combine(x_r, hn, ws, wn, bsn, rst, cb):
    n, din = x_r.shape
    dout = ws.shape[1]
    nt = n // 1000

    def body(x_ref, h_ref, ws_ref, wn_ref, b_ref, r_ref, cb_ref,
             y_ref, s_ref, q_ref):
        i = pl.program_id(0)
        sage = (_DOT(x_ref[...], ws_ref[...]) + _DOT(h_ref[...], wn_ref[...])
                + b_ref[...])
        y = jnp.maximum(jnp.maximum(sage, 0.0) + r_ref[...] + cb_ref[...],
                        0.0)
        y_ref[...] = y

        @pl.when(i == 0)
        def _():
            s_ref[...] = jnp.zeros_like(s_ref)
            q_ref[...] = jnp.zeros_like(q_ref)

        s_ref[...] += jnp.sum(y, axis=0, keepdims=True)
        q_ref[...] += jnp.sum(y * y, axis=0, keepdims=True)

    full = lambda shape: pl.BlockSpec(shape, lambda i: (0, 0))
    return pl.pallas_call(
        body,
        grid=(nt,),
        in_specs=[pl.BlockSpec((1000, din), lambda i: (i, 0)),
                  pl.BlockSpec((1000, din), lambda i: (i, 0)),
                  full(ws.shape), full(wn.shape), full(bsn.shape),
                  pl.BlockSpec((1000, dout), lambda i: (i, 0)),
                  full(cb.shape)],
        out_specs=[pl.BlockSpec((1000, dout), lambda i: (i, 0)),
                   full((1, dout)), full((1, dout))],
        out_shape=[jax.ShapeDtypeStruct((n, dout), jnp.float32),
                   jax.ShapeDtypeStruct((1, dout), jnp.float32),
                   jax.ShapeDtypeStruct((1, dout), jnp.float32)],
        interpret=_INTERP,
    )(x_r, hn, ws, wn, bsn, rst, cb)


def _normalize_pad(y, s, t, dpad):
    n, dout = y.shape
    nt = n // 1000

    def body(y_ref, s_ref, t_ref, o_ref):
        v = y_ref[...] * s_ref[...] + t_ref[...]
        if dpad > dout:
            v = jnp.concatenate(
                [v, jnp.zeros((v.shape[0], dpad - dout), jnp.float32)],
                axis=-1)
        o_ref[...] = v

    full = lambda shape: pl.BlockSpec(shape, lambda i: (0, 0))
    return pl.pallas_call(
        body,
        grid=(nt,),
        in_specs=[pl.BlockSpec((1000, dout), lambda i: (i, 0)),
                  full((1, dout)), full((1, dout))],
        out_specs=pl.BlockSpec((1000, dpad), lambda i: (i, 0)),
        out_shape=jax.ShapeDtypeStruct((n, dpad), jnp.float32),
        interpret=_INTERP,
    )(y, s, t)


def _colmax(y):
    n, d = y.shape
    nt = n // 1000

    def body(y_ref, m_ref):
        i = pl.program_id(0)

        @pl.when(i == 0)
        def _():
            m_ref[...] = jnp.full_like(m_ref, -jnp.inf)

        m_ref[...] = jnp.maximum(m_ref[...],
                                 jnp.max(y_ref[...], axis=0, keepdims=True))

    return pl.pallas_call(
        body,
        grid=(nt,),
        in_specs=[pl.BlockSpec((1000, d), lambda i: (i, 0))],
        out_specs=pl.BlockSpec((1, d), lambda i: (0, 0)),
        out_shape=jax.ShapeDtypeStruct((1, d), jnp.float32),
        interpret=_INTERP,
    )(y)


def _head(lf, rf, w1a, w1b, b1, w2, b2, w3, b3):
    def body(lf_ref, rf_ref, w1a_ref, w1b_ref, b1_ref, w2_ref, b2_ref,
             w3_ref, b3_ref, o_ref):
        h1 = jnp.maximum(_DOT(lf_ref[...], w1a_ref[...])
                         + _DOT(rf_ref[...], w1b_ref[...]) + b1_ref[...], 0.0)
        h2 = jnp.maximum(_DOT(h1, w2_ref[...]) + b2_ref[...], 0.0)
        o_ref[...] = _DOT(h2, w3_ref[...]) + b3_ref[...]

    return pl.pallas_call(
        body,
        out_shape=jax.ShapeDtypeStruct((1, 1), jnp.float32),
        interpret=_INTERP,
    )(lf, rf, w1a, w1b, b1, w2, b2, w3, b3)


_MESH = plsc.VectorSubcoreMesh(core_axis_name="c", subcore_axis_name="s")


def _sc_gather_rows(x_pad, idx3):
    nw, nblk, bw = idx3.shape
    dinp = x_pad.shape[1]

    @functools.partial(
        pl.kernel,
        out_type=jax.ShapeDtypeStruct((nw * nblk * bw, dinp), jnp.float32),
        mesh=_MESH,
        scratch_types=[pltpu.VMEM((nblk, bw), jnp.int32),
                       pltpu.VMEM((bw, dinp), jnp.float32),
                       pltpu.VMEM((bw, dinp), jnp.float32),
                       pltpu.SemaphoreType.DMA,
                       pltpu.SemaphoreType.DMA],
    )
    def k(x_hbm, idx_hbm, out_hbm, idxb, rb0, rb1, s0, s1):
        wid = lax.axis_index("s") * 2 + lax.axis_index("c")
        pltpu.sync_copy(idx_hbm.at[wid], idxb)
        base = wid * (nblk * bw)
        bufs, sems = (rb0, rb1), (s0, s1)
        cps = {0: pltpu.async_copy(x_hbm.at[idxb.at[0]], rb0, s0)}
        for j in range(nblk):
            p = j % 2
            if j + 1 < nblk:
                cps[(j + 1) % 2] = pltpu.async_copy(
                    x_hbm.at[idxb.at[j + 1]], bufs[(j + 1) % 2],
                    sems[(j + 1) % 2])
            cps[p].wait()
            pltpu.sync_copy(bufs[p], out_hbm.at[pl.ds(base + j * bw, bw)])

    return k(x_pad, idx3)


def _sc_segment_sum(x_flat, srcf3, dst3, n_grp):
    _, nsb, sbe = srcf3.shape
    nbl = sbe // 128
    nlp = NL + 16
    nrow = nlp // 16

    @functools.partial(
        pl.kernel,
        out_type=jax.ShapeDtypeStruct((nlp, n_grp, 64), jnp.float32),
        mesh=_MESH,
        scratch_types=[pltpu.VMEM((sbe,), jnp.int32),
                       pltpu.VMEM((nbl, 128), jnp.int32),
                       pltpu.VMEM((128,), jnp.int32),
                       pltpu.VMEM((128,), jnp.int32),
                       pltpu.VMEM((128, 64), jnp.float32),
                       pltpu.VMEM((128, 64), jnp.float32),
                       pltpu.VMEM_SHARED((nlp, 64), jnp.float32),
                       pltpu.SemaphoreType.DMA,
                       pltpu.SemaphoreType.DMA],
        compiler_params=pltpu.CompilerParams(use_tc_tiling_on_sc=False),
    )
    def k(x_hbm, src_hbm, dst_hbm, out_hbm,
          srcf, dstb, gx0, gx1, rb0, rb1, acc, s0, s1):
        cid = lax.axis_index("c")
        sid = lax.axis_index("s")
        zv = jnp.zeros((16,), jnp.float32)

        def zfill(i, _):
            for q in range(4):
                rb0[i, pl.ds(16 * q, 16)] = zv
            return 0

        lo = sid * nrow
        gxs, bufs, sems = (gx0, gx1), (rb0, rb1), (s0, s1)
        for gi in range(n_grp // 2):
            g = gi * 2 + cid
            lax.fori_loop(0, 128, zfill, 0)
            nfull = nrow // 128
            for j in range(nfull):
                pltpu.sync_copy(rb0, acc.at[pl.ds(lo + j * 128, 128)])
            rem = nrow - nfull * 128
            if rem:
                pltpu.sync_copy(rb0.at[pl.ds(0, rem)],
                                acc.at[pl.ds(lo + nfull * 128, rem)])
            plsc.subcore_barrier()

            def sbody(sb, _):
                pltpu.sync_copy(src_hbm.at[sid, sb], srcf)
                pltpu.sync_copy(dst_hbm.at[sid, sb], dstb)

                def fire(j, p):
                    def gfill(kk, _):
                        sv = srcf[pl.ds(j * 128 + kk * 16, 16)]
                        gxs[p][pl.ds(kk * 16, 16)] = sv * n_grp + g
                        return 0

                    lax.fori_loop(0, 8, gfill, 0)
                    return pltpu.async_copy(x_hbm.at[gxs[p]], bufs[p],
                                            sems[p])

                cps = {0: fire(0, 0)}
                for j in range(nbl):
                    p = j % 2
                    cps[p].wait()
                    if j + 1 < nbl:
                        cps[(j + 1) % 2] = fire(j + 1, (j + 1) % 2)
                    pltpu.sync_copy(bufs[p], acc.at[dstb.at[j]], add=True)
                return 0

            lax.fori_loop(0, nsb, sbody, 0)
            plsc.subcore_barrier()
            nfull = nrow // 128
            for j in range(nfull):
                pltpu.sync_copy(acc.at[pl.ds(lo + j * 128, 128)],
                                out_hbm.at[pl.ds(lo + j * 128, 128), g])
            rem = nrow - nfull * 128
            if rem:
                pltpu.sync_copy(acc.at[pl.ds(lo + nfull * 128, rem)],
                                out_hbm.at[pl.ds(lo + nfull * 128, rem), g])
            plsc.subcore_barrier()

    return k(x_flat, srcf3, dst3)


def _sc_gat_pass1(el2, er2, m16, srcp1, dstp1, nerrp):
    nt = nerrp // 16
    nv = nt // 16

    @functools.partial(
        pl.kernel,
        out_type=[jax.ShapeDtypeStruct((H, nerrp), jnp.float32),
                  jax.ShapeDtypeStruct((2, 384, 64), jnp.float32)],
        mesh=_MESH,
        scratch_types=[pltpu.VMEM((NR + 16, 4), jnp.float32),
                       pltpu.VMEM((NR + 16, 4), jnp.float32),
                       pltpu.VMEM((128,), jnp.float32),
                       pltpu.VMEM((nt,), jnp.int32),
                       pltpu.VMEM((nt,), jnp.int32),
                       pltpu.VMEM((4, nt), jnp.float32),
                       pltpu.VMEM((384, 64), jnp.float32),
                       pltpu.VMEM((3, 128), jnp.int32),
                       pltpu.VMEM_SHARED((384, 64), jnp.float32)],
        compiler_params=pltpu.CompilerParams(use_tc_tiling_on_sc=False, needs_layout_passes=False),
    )
    def k(el_hbm, er_hbm, m_hbm, src_hbm, dst_hbm, ex_hbm, den_hbm,
          elb, erb, mb, srcf, dstf, exb, den, ident, den_sh):
        cid = lax.axis_index("c")
        sid = lax.axis_index("s")
        pltpu.sync_copy(el_hbm.at[cid], elb.at[pl.ds(0, NR)])
        pltpu.sync_copy(er_hbm.at[cid], erb.at[pl.ds(0, NR)])
        pltpu.sync_copy(m_hbm, mb)
        pltpu.sync_copy(src_hbm.at[sid], srcf)
        pltpu.sync_copy(dst_hbm.at[sid], dstf)
        zv = jnp.zeros((16,), jnp.float32)
        iot = lax.iota(jnp.int32, 16)

        def dfill(i, _):
            for q in range(4):
                den[i, pl.ds(16 * q, 16)] = zv
            return 0

        lax.fori_loop(0, 384, dfill, 0)

        def ifill(i, _):
            ident[i >> 3, pl.ds((i & 7) * 16, 16)] = iot + i * 16
            return 0

        lax.fori_loop(0, 24, ifill, 0)
        pltpu.sync_copy(den.at[pl.ds(0, 24)], den_sh.at[pl.ds(sid * 24, 24)])
        plsc.subcore_barrier()

        def ebody(j, _):
            sv = srcf[pl.ds(16 * j, 16)]
            dv = dstf[pl.ds(16 * j, 16)]
            for hh in range(4):
                hv = jnp.full((16,), hh, jnp.int32)
                li = plsc.load_gather(elb, [sv, hv])
                ri = plsc.load_gather(erb, [dv, hv])
                e = li + ri
                e = jnp.where(e >= 0, e, 0.2 * e)
                mv = mb[pl.ds((4 * cid + hh) * 16, 16)]
                ex = jnp.exp(e - mv)
                exb[hh, pl.ds(16 * j, 16)] = ex
                flat = dv * 4 + hh
                plsc.addupdate_scatter(
                    den, [lax.shift_right_logical(flat, 6), flat & 63], ex)
            return 0

        lax.fori_loop(0, nv, ebody, 0)
        for j in range(3):
            pltpu.sync_copy(den.at[pl.ds(128 * j, 128)],
                            den_sh.at[ident.at[j]], add=True)
        plsc.subcore_barrier()
        for hh in range(4):
            pltpu.sync_copy(exb.at[hh],
                            ex_hbm.at[4 * cid + hh, pl.ds(sid * nt, nt)])
        pltpu.sync_copy(den_sh.at[pl.ds(sid * 24, 24)],
                        den_hbm.at[cid, pl.ds(sid * 24, 24)])

    return k(el2, er2, m16, srcp1, dstp1)


def _sc_gat_pass2(feat_flat, den_pad, ex, srcp1, dst2p, qp):
    nerrp = ex.shape[1]
    nt = nerrp // 16
    nbl = nt // 128
    fw = feat_flat.shape[1]
    hq16 = qp // 16

    @functools.partial(
        pl.kernel,
        out_type=jax.ShapeDtypeStruct((4, 5120, qp), jnp.float32),
        mesh=_MESH,
        scratch_types=[pltpu.VMEM((nt,), jnp.int32),
                       pltpu.VMEM((nt,), jnp.int32),
                       pltpu.VMEM((20, 128), jnp.int32),
                       pltpu.VMEM((8 * nt,), jnp.float32),
                       pltpu.VMEM((128, 16), jnp.float32),
                       pltpu.VMEM((16, fw), jnp.float32),
                       pltpu.VMEM((16, fw), jnp.float32),
                       pltpu.VMEM((128, qp), jnp.float32),
                       pltpu.VMEM_SHARED((5120, qp), jnp.float32),
                       pltpu.SemaphoreType.DMA,
                       pltpu.SemaphoreType.DMA,
                       pltpu.SemaphoreType.DMA],
        compiler_params=pltpu.CompilerParams(use_tc_tiling_on_sc=False, needs_layout_passes=False),
    )
    def k(f_hbm, den_hbm, ex_hbm, src_hbm, dst2_hbm, out_hbm,
          srcf, srcg, dst2d, exb, denb, fb0, fb1, wbuf, rst_sh, sA, s0, s1):
        cid = lax.axis_index("c")
        sid = lax.axis_index("s")
        pltpu.sync_copy(src_hbm.at[sid], srcf)
        pltpu.sync_copy(dst2_hbm.at[sid], dst2d)
        iot = lax.iota(jnp.int32, 16)
        zv = jnp.zeros((16,), jnp.float32)
        fbs, sems = (fb0, fb1), (s0, s1)
        lo = sid * 320

        for h in range(H):
            pltpu.sync_copy(ex_hbm.at[h, pl.ds(sid * nt, nt)],
                            exb.at[pl.ds(h * nt, nt)])

        def ablock(jb, _):
            pltpu.async_copy(den_hbm.at[dst2d.at[jb]], denb, sA).wait()

            def alph(kk, _):
                idxv = iot + 16 * kk
                for h in range(H):
                    hv = jnp.full((16,), h, jnp.int32)
                    dg = plsc.load_gather(denb, [idxv, hv])
                    o = h * nt + jb * 128 + 16 * kk
                    exb[pl.ds(o, 16)] = (exb[pl.ds(o, 16)]
                                         / ((dg + 1e-9) * 8.0))
                return 0

            lax.fori_loop(0, 8, alph, 0)
            return 0

        lax.fori_loop(0, nbl, ablock, 0)

        def zfill(i, _):
            iv = jnp.full((16,), i, jnp.int32)
            for q in range(hq16):
                plsc.store_scatter(wbuf, [iv, iot + 16 * q], zv)
            return 0

        for gi in range(2):
            qg = gi * 2 + cid

            def soff(j, _):
                srcg[pl.ds(16 * j, 16)] = srcf[pl.ds(16 * j, 16)] + NR * qg
                return 0

            lax.fori_loop(0, nt // 16, soff, 0)
            lax.fori_loop(0, 128, zfill, 0)
            pltpu.sync_copy(wbuf, rst_sh.at[pl.ds(lo, 128)])
            pltpu.sync_copy(wbuf, rst_sh.at[pl.ds(lo + 128, 128)])
            pltpu.sync_copy(wbuf.at[pl.ds(0, 64)],
                            rst_sh.at[pl.ds(lo + 256, 64)])
            plsc.subcore_barrier()

            def sub(jb, sq, p):
                def edge(ee, _):
                    col = sq * 16 + ee
                    colv = jnp.full((16,), col, jnp.int32)
                    gcol = jb * 128 + col
                    avs = [plsc.load_gather(
                        exb, [jnp.full((16,), h * nt + gcol, jnp.int32)])
                        for h in range(H)]
                    eev = jnp.full((16,), ee, jnp.int32)
                    for q in range(hq16):
                        acc = avs[0] * plsc.load_gather(
                            fbs[p], [eev, iot + q * 16])
                        for h in range(1, H):
                            acc += avs[h] * plsc.load_gather(
                                fbs[p], [eev, iot + h * qp + q * 16])
                        plsc.store_scatter(wbuf, [colv, iot + q * 16], acc)
                    return 0

                lax.fori_loop(0, 16, edge, 0)

            def fire(jb, sq, p):
                sqc = jnp.minimum(sq, 7)
                return pltpu.async_copy(
                    f_hbm.at[srcg.at[pl.ds(jb * 128 + sqc * 16, 16)]],
                    fbs[p], sems[p])

            def drain(p):
                pltpu.make_async_copy(f_hbm.at[pl.ds(0, 16)], fbs[p],
                                      sems[p]).wait()

            def block(jb, _):
                fire(jb, 0, 0)

                def pair(qq, _):
                    fire(jb, 2 * qq + 1, 1)
                    drain(0)
                    sub(jb, 2 * qq, 0)
                    fire(jb, 2 * qq + 2, 0)
                    drain(1)
                    sub(jb, 2 * qq + 1, 1)
                    return 0

                lax.fori_loop(0, 4, pair, 0)
                drain(0)
                pltpu.sync_copy(wbuf, rst_sh.at[dst2d.at[jb]], add=True)
                return 0

            lax.fori_loop(0, nbl, block, 0)
            plsc.subcore_barrier()
            pltpu.sync_copy(rst_sh.at[pl.ds(lo, 128)],
                            out_hbm.at[qg, pl.ds(lo, 128)])
            pltpu.sync_copy(rst_sh.at[pl.ds(lo + 128, 128)],
                            out_hbm.at[qg, pl.ds(lo + 128, 128)])
            pltpu.sync_copy(rst_sh.at[pl.ds(lo + 256, 64)],
                            out_hbm.at[qg, pl.ds(lo + 256, 64)])
            plsc.subcore_barrier()

    return k(feat_flat, den_pad, ex, srcp1, dst2p)


def _sage_gather(x_pad, gidx):
    return x_pad[gidx]


def _gin_agg(x_pad, src, dst):
    return jax.ops.segment_sum(x_pad[src], dst, num_segments=NL)


def _gat_edges(el, er, mvec, feat_a, feat_b, src, dst, dout, halfp):
    half = dout // 2
    e = el[src] + er[dst]
    e = jnp.where(e >= 0, e, 0.2 * e)
    ex = jnp.exp(e - mvec[None, :])
    den = jax.ops.segment_sum(ex, dst, num_segments=NR)
    alpha = ex / (den[dst] + 1e-9) / H
    feat = jnp.concatenate([feat_a[:, :, :half], feat_b[:, :, :half]],
                           axis=-1)
    w = jnp.sum(feat[src] * alpha[:, :, None], axis=1)
    return jax.ops.segment_sum(w, dst, num_segments=NR)


def _finalize_bn(ssum, ssq, gamma, beta, n):
    mean = ssum / n
    var = ssq / n - mean * mean
    s = gamma.reshape(1, -1) / jnp.sqrt(var + 1e-5)
    t = beta.reshape(1, -1) - mean * s
    return s, t


def kernel(x_ligand, x_residue, bond_edge_index, rr_edge_index,
           lr_neighbors, params):
    src_b, dst_b = bond_edge_index[0], bond_edge_index[1]
    src_r, dst_r = rr_edge_index[0], rr_edge_index[1]
    lrn_t = jnp.swapaxes(lr_neighbors, 0, 1).reshape(-1)

    ebp = 16 * 10 * 2048
    srcb3 = jnp.pad(src_b, (0, ebp - EB)).reshape(16, 10, 2048)
    dstb3 = jnp.pad(dst_b, (0, ebp - EB),
                    constant_values=NL).reshape(16, 10, 16, 128)
    idx3 = jnp.pad(lrn_t, (0, 32 * 20 * 128 - KN * NR)).reshape(32, 20, 128)
    errp = 16 * 2560
    srcp1 = jnp.pad(src_r, (0, errp - ERR)).reshape(16, 2560)
    dstp1 = jnp.pad(dst_r, (0, errp - ERR), constant_values=NR).reshape(16, 2560)
    dst2p = dstp1.reshape(16, 20, 128)

    dims = [(108, 108), (108, 216), (216, 432)]
    dinp0 = _pad_up(108, 64)
    x_l = jnp.pad(x_ligand, ((0, 0), (0, dinp0 - 108)))
    x_r = x_residue

    for li, lp in enumerate(params["layers"]):
        din, dout = dims[li]
        dinp = _pad_up(din, 64)
        doutp = _pad_up(dout, 64)
        qp = doutp // 4

        n_grp = dinp // 64
        agg = _sc_segment_sum(x_l.reshape(NL * n_grp, 64), srcb3, dstb3,
                              n_grp)[:NL].reshape(NL, dinp)
        w1p = jnp.pad(lp["gin"]["l1"]["W"], ((0, dinp - din), (0, 0)))
        y_l, sl, ql = _gin_mlp(x_l, agg, w1p,
                               lp["gin"]["l1"]["b"].reshape(1, -1),
                               lp["gin"]["l2"]["W"],
                               lp["gin"]["l2"]["b"].reshape(1, -1))

        f0, f1, f2, f3s, el, er, ml, mr = _gat_feat(x_r, lp["gat"]["fc"],
                                                    lp["gat"]["attn_l"],
                                                    lp["gat"]["attn_r"], qp)
        msum = (ml + mr).reshape(-1)
        mvec = jnp.where(msum >= 0, msum, 0.2 * msum)
        m16 = jnp.repeat(mvec, 16)
        el2 = el.reshape(NR, 2, 4).transpose(1, 0, 2)
        er2 = er.reshape(NR, 2, 4).transpose(1, 0, 2)
        ex, den2 = _sc_gat_pass1(el2, er2, m16, srcp1, dstp1, errp)
        den = (den2.reshape(2, 24576)[:, :NR * 4].reshape(2, NR, 4)
               .transpose(1, 0, 2).reshape(NR, H))
        den_pad = jnp.pad(den, ((0, 5120 - NR), (0, 8)))
        featf = jnp.stack([f0, f1, f2, f3s]).reshape(4 * NR, H * qp)
        rst4 = _sc_gat_pass2(featf, den_pad, ex, srcp1, dst2p, qp)
        rst = jnp.concatenate([rst4[q][:NR] for q in range(4)],
                              axis=1)[:, :dout]

        neigh = _sc_gather_rows(x_l, idx3)
        wihp = jnp.pad(lp["sage"]["W_ih"], ((0, dinp - din), (0, 0)))
        bsum = (lp["sage"]["b_ih"] + lp["sage"]["b_hh"]).reshape(1, -1)
        hn = _lstm(neigh, wihp, lp["sage"]["W_hh"], bsum)

        cb = jnp.mean(lp["gat"]["bias"], axis=0).reshape(1, -1)
        bsn = (lp["sage"]["fc_self"]["b"]
               + lp["sage"]["fc_neigh"]["b"]).reshape(1, -1)
        y_r, sr, qr = _sage_combine(x_r, hn, lp["sage"]["fc_self"]["W"],
                                    lp["sage"]["fc_neigh"]["W"], bsn, rst, cb)

        s_l, t_l = _finalize_bn(sl, ql, lp["bn_l"]["gamma"],
                                lp["bn_l"]["beta"], NL)
        s_r, t_r = _finalize_bn(sr, qr, lp["bn_r"]["gamma"],
                                lp["bn_r"]["beta"], NR)
        if li < 2:
            dnext = _pad_up(dims[li + 1][0], 64)
            x_l = _normalize_pad(y_l, s_l, t_l, dnext)
            x_r = _normalize_pad(y_r, s_r, t_r, dout)
        else:
            lig_feat = _colmax(y_l) * s_l + t_l
            res_feat = _colmax(y_r) * s_r + t_r

    x3 = _head(lig_feat, res_feat,
               params["fc01"]["W"][:432], params["fc01"]["W"][432:],
               params["fc01"]["b"].reshape(1, -1),
               params["fc02"]["W"], params["fc02"]["b"].reshape(1, -1),
               params["fc03"]["W"], params["fc03"]["b"].reshape(1, -1))
    return (lig_feat, res_feat, x3)
